# Initial kernel scaffold; baseline (speedup 1.0000x reference)
import jax
import jax.numpy as jnp
from jax.experimental import pallas as pl


def kernel(hidden_states, cos, sin, q_a_proj, q_a_ln, q_b_proj, kv_a_proj, kv_a_ln, kv_b_proj, o_proj, idx_wq_b, idx_wk, idx_knorm_w, idx_knorm_b, idx_wproj):
    raise NotImplementedError("write your pallas kernel here")


# speedup vs baseline: 9.5327x; 9.5327x over previous
import jax
import jax.numpy as jnp
from jax.experimental import pallas as pl

B, S, HID = 1, 2048, 2048
H, NOPE, ROPE, VD = 16, 128, 64, 128
QKD = NOPE + ROPE
QLORA, KVLORA = 1536, 512
IH, ID, TOPK = 16, 128, 512
NEG = -1e30

BQ = 256
NB = S // BQ
HI = jax.lax.Precision.HIGHEST
F32 = jnp.float32
BF16 = jnp.bfloat16


def _mm(a, b, prec=HI):
    return jax.lax.dot_general(a, b, (((1,), (0,)), ((), ())),
                               preferred_element_type=F32, precision=prec)


def _mm_t(a, b, prec=HI):
    return jax.lax.dot_general(a, b, (((1,), (1,)), ((), ())),
                               preferred_element_type=F32, precision=prec)


def _rope(x, cos, sin):
    half = x.shape[-1] // 2
    x1 = x[..., :half]
    x2 = x[..., half:]
    rot = jnp.concatenate([-x2, x1], axis=-1)
    return x * cos + rot * sin


def _prolog_kernel(x_ref, cos_ref, sin_ref, qaw_ref, qaln_ref, kvaw_ref,
                   kvaln_ref, ikw_ref, iknw_ref, iknb_ref,
                   qres_ref, ik_ref, iw_ref, ckv_ref, kpe_ref):
    x = x_ref[...]
    xb = x.astype(BF16)
    cos = cos_ref[...]
    sin = sin_ref[...]

    qa = _mm(xb, qaw_ref[...], prec=None)
    qres = qa * jax.lax.rsqrt(jnp.mean(qa * qa, axis=-1, keepdims=True)
                              + 1e-6) * qaln_ref[...]
    qres_ref[...] = qres

    ikw = _mm(xb, ikw_ref[...], prec=None)
    ikr = ikw[:, :ID]
    iw_ref[...] = ikw[:, ID:ID + IH]
    m = jnp.mean(ikr, axis=-1, keepdims=True)
    v = jnp.mean((ikr - m) ** 2, axis=-1, keepdims=True)
    ikn = (ikr - m) * jax.lax.rsqrt(v + 1e-6) * iknw_ref[...] + iknb_ref[...]
    ik_ref[...] = jnp.concatenate(
        [_rope(ikn[:, :ROPE], cos, sin), ikn[:, ROPE:]], axis=-1)

    kv = _mm(xb, kvaw_ref[...], prec=None)
    ckv = kv[:, :KVLORA]
    ckvn = ckv * jax.lax.rsqrt(jnp.mean(ckv * ckv, axis=-1, keepdims=True)
                               + 1e-6) * kvaln_ref[...]
    ckv_ref[...] = ckvn.astype(BF16)
    kpe = _rope(kv[:, KVLORA:], cos, sin).astype(BF16)
    kpe_ref[...] = jnp.concatenate(
        [kpe, jnp.zeros((BQ, NOPE - ROPE), BF16)], axis=-1)


def _iqsum_kernel(qres_ref, iw_ref, cos_ref, sin_ref, wqb_ref, qsum_ref):
    iq = _mm(qres_ref[...].astype(BF16), wqb_ref[...], prec=None)
    iw = iw_ref[...]
    cos = cos_ref[...]
    sin = sin_ref[...]
    scale = jnp.float32(ID ** -0.5)
    qsum = jnp.zeros((BQ, ID), F32)
    for h in range(IH):
        iqh = iq[:, h * ID:(h + 1) * ID]
        iqh = jnp.concatenate([_rope(iqh[:, :ROPE], cos, sin), iqh[:, ROPE:]],
                              axis=-1)
        qsum = qsum + (iqh * iw[:, h:h + 1]) * scale
    qsum_ref[...] = qsum


def _score_mask_kernel(qsum_ref, ik_ref, mask_ref):
    i = pl.program_id(0)
    s = _mm_t(qsum_ref[...].astype(BF16), ik_ref[...].astype(BF16),
              prec=None)
    rows = jax.lax.broadcasted_iota(jnp.int32, (BQ, S), 0) + i * BQ
    cols = jax.lax.broadcasted_iota(jnp.int32, (BQ, S), 1)
    causal = cols <= rows
    s = jnp.where(causal, s, NEG)

    lo = jnp.min(jnp.where(causal, s, jnp.float32(1e30)), axis=1,
                 keepdims=True)
    hi = jnp.max(s, axis=1, keepdims=True) + 1.0

    def body(_, carry):
        lo, hi = carry
        mid = 0.5 * (lo + hi)
        cnt = jnp.sum((s >= mid).astype(F32), axis=1, keepdims=True)
        ge = cnt >= TOPK
        return jnp.where(ge, mid, lo), jnp.where(ge, hi, mid)

    lo, hi = jax.lax.fori_loop(0, 44, body, (lo, hi))
    active = (s >= lo) | (causal & (rows < TOPK))
    mask_ref[...] = active.astype(jnp.int8)


def _expand_kernel(qres_ref, ckv_ref, cos_ref, sin_ref, qbw_ref, kvbw_ref,
                   qn_ref, qp_ref, kn_ref, v_ref):
    cos = cos_ref[...]
    sin = sin_ref[...]
    q = _mm(qres_ref[...].astype(BF16), qbw_ref[...], prec=None)
    qn_ref[...] = q[:, :H * NOPE].astype(BF16)
    zero = jnp.zeros((BQ, NOPE - ROPE), BF16)
    for h in range(H):
        pe = q[:, H * NOPE + h * ROPE:H * NOPE + (h + 1) * ROPE]
        qp_ref[:, h * NOPE:h * NOPE + ROPE] = _rope(pe, cos, sin).astype(BF16)
        qp_ref[:, h * NOPE + ROPE:(h + 1) * NOPE] = zero
    kvb = _mm(ckv_ref[...], kvbw_ref[...], prec=None)
    kn_ref[...] = kvb[:, :H * NOPE].astype(BF16)
    v_ref[...] = kvb[:, H * NOPE:].astype(BF16)


def _flash_kernel(qn_ref, qp_ref, kn_ref, kpe_ref, v_ref, mask_ref, out_ref):
    i = pl.program_id(1)
    q = qn_ref[...]
    qp = qp_ref[...]
    scale = jnp.float32(QKD ** -0.5)

    def body(j, carry):
        m, l, acc = carry
        kb = kn_ref[pl.ds(j * BQ, BQ), :]
        kpb = kpe_ref[pl.ds(j * BQ, BQ), :]
        vb = v_ref[pl.ds(j * BQ, BQ), :]
        s = (_mm_t(q, kb, prec=None) + _mm_t(qp, kpb, prec=None)) * scale
        mb = mask_ref[:, pl.ds(j * BQ, BQ)]
        s = jnp.where(mb != 0, s, NEG)
        mnew = jnp.maximum(m, jnp.max(s, axis=1, keepdims=True))
        p = jnp.exp(s - mnew)
        alpha = jnp.exp(m - mnew)
        l = l * alpha + jnp.sum(p, axis=1, keepdims=True)
        acc = acc * alpha + _mm(p.astype(BF16), vb, prec=None)
        return mnew, l, acc

    m0 = jnp.full((BQ, 1), NEG, F32)
    l0 = jnp.zeros((BQ, 1), F32)
    a0 = jnp.zeros((BQ, VD), F32)
    m, l, acc = jax.lax.fori_loop(0, i + 1, body, (m0, l0, a0))
    out_ref[...] = (acc / l).astype(BF16)


def _oproj_kernel(a_ref, w_ref, out_ref):
    out_ref[...] = _mm(a_ref[...], w_ref[...], prec=None)


def kernel(hidden_states, cos, sin, q_a_proj, q_a_ln, q_b_proj, kv_a_proj,
           kv_a_ln, kv_b_proj, o_proj, idx_wq_b, idx_wk, idx_knorm_w,
           idx_knorm_b, idx_wproj):
    x = hidden_states[0]
    cos2 = cos[0, :, :ROPE]
    sin2 = sin[0, :, :ROPE]
    ikw_cat = jnp.concatenate([idx_wk, idx_wproj], axis=1)

    qb3 = q_b_proj.reshape(QLORA, H, QKD)
    qb_perm = jnp.concatenate([qb3[:, :, :NOPE].reshape(QLORA, H * NOPE),
                               qb3[:, :, NOPE:].reshape(QLORA, H * ROPE)],
                              axis=1).astype(BF16)
    kvb3 = kv_b_proj.reshape(KVLORA, H, NOPE + VD)
    kvb_perm = jnp.concatenate([kvb3[:, :, :NOPE].reshape(KVLORA, H * NOPE),
                                kvb3[:, :, NOPE:].reshape(KVLORA, H * VD)],
                               axis=1).astype(BF16)

    row = lambda blk: pl.BlockSpec(blk, lambda i: (i,) + (0,) * (len(blk) - 1))
    full = lambda shp: pl.BlockSpec(shp, lambda i: (0,) * len(shp))

    qres, ik, iw, ckv, kpe = pl.pallas_call(
        _prolog_kernel,
        grid=(NB,),
        in_specs=[row((BQ, HID)), row((BQ, ROPE)), row((BQ, ROPE)),
                  full((HID, QLORA)), full((1, QLORA)), full((HID, KVLORA + ROPE)),
                  full((1, KVLORA)), full((HID, ID + IH)), full((1, ID)),
                  full((1, ID))],
        out_specs=[row((BQ, QLORA)), row((BQ, ID)), row((BQ, IH)),
                   row((BQ, KVLORA)), row((BQ, NOPE))],
        out_shape=[jax.ShapeDtypeStruct((S, QLORA), F32),
                   jax.ShapeDtypeStruct((S, ID), F32),
                   jax.ShapeDtypeStruct((S, IH), F32),
                   jax.ShapeDtypeStruct((S, KVLORA), BF16),
                   jax.ShapeDtypeStruct((S, NOPE), BF16)],
    )(x, cos2, sin2, q_a_proj.astype(BF16), q_a_ln[None],
      kv_a_proj.astype(BF16), kv_a_ln[None], ikw_cat.astype(BF16),
      idx_knorm_w[None], idx_knorm_b[None])

    qsum = pl.pallas_call(
        _iqsum_kernel,
        grid=(NB,),
        in_specs=[row((BQ, QLORA)), row((BQ, IH)), row((BQ, ROPE)),
                  row((BQ, ROPE)), full((QLORA, IH * ID))],
        out_specs=row((BQ, ID)),
        out_shape=jax.ShapeDtypeStruct((S, ID), F32),
    )(qres, iw, cos2, sin2, idx_wq_b.astype(BF16))

    mask = pl.pallas_call(
        _score_mask_kernel,
        grid=(NB,),
        in_specs=[row((BQ, ID)), full((S, ID))],
        out_specs=row((BQ, S)),
        out_shape=jax.ShapeDtypeStruct((S, S), jnp.int8),
    )(qsum, ik)

    qn, qp, kn, v = pl.pallas_call(
        _expand_kernel,
        grid=(NB,),
        in_specs=[row((BQ, QLORA)), row((BQ, KVLORA)), row((BQ, ROPE)),
                  row((BQ, ROPE)), full((QLORA, H * QKD)),
                  full((KVLORA, H * (NOPE + VD)))],
        out_specs=[row((BQ, H * NOPE)), row((BQ, H * NOPE)),
                   row((BQ, H * NOPE)), row((BQ, H * VD))],
        out_shape=[jax.ShapeDtypeStruct((S, H * NOPE), BF16),
                   jax.ShapeDtypeStruct((S, H * NOPE), BF16),
                   jax.ShapeDtypeStruct((S, H * NOPE), BF16),
                   jax.ShapeDtypeStruct((S, H * VD), BF16)],
    )(qres, ckv, cos2, sin2, qb_perm, kvb_perm)

    attn = pl.pallas_call(
        _flash_kernel,
        grid=(H, NB),
        in_specs=[
            pl.BlockSpec((BQ, NOPE), lambda h, i: (i, h)),
            pl.BlockSpec((BQ, NOPE), lambda h, i: (i, h)),
            pl.BlockSpec((S, NOPE), lambda h, i: (0, h)),
            pl.BlockSpec((S, NOPE), lambda h, i: (0, 0)),
            pl.BlockSpec((S, VD), lambda h, i: (0, h)),
            pl.BlockSpec((BQ, S), lambda h, i: (i, 0)),
        ],
        out_specs=pl.BlockSpec((BQ, VD), lambda h, i: (i, h)),
        out_shape=jax.ShapeDtypeStruct((S, H * VD), BF16),
    )(qn, qp, kn, kpe, v, mask)

    out = pl.pallas_call(
        _oproj_kernel,
        grid=(NB,),
        in_specs=[row((BQ, H * VD)), full((H * VD, HID))],
        out_specs=row((BQ, HID)),
        out_shape=jax.ShapeDtypeStruct((S, HID), F32),
    )(attn, o_proj.astype(BF16))

    return out[None]
